# baseline (speedup 1.0000x reference)
import functools

import jax
import jax.numpy as jnp
from jax import lax
from jax.experimental import pallas as pl
from jax.experimental.pallas import tpu as pltpu
from jax.experimental.pallas import tpu_sc as plsc

NC = 2
NS = 16
L = 16
NW = NC * NS
SCH = 4096
GCH = 64
QCAP = SCH + 2 * GCH


@functools.lru_cache(maxsize=None)
def _make_sc_kernel(nsc, outf, b):
    rpt = outf // NW
    mesh = plsc.VectorSubcoreMesh(core_axis_name="c", subcore_axis_name="s")

    @functools.partial(
        pl.kernel,
        out_type=jax.ShapeDtypeStruct((outf, b), jnp.float32),
        mesh=mesh,
        compiler_params=pltpu.CompilerParams(needs_layout_passes=False),
        scratch_types=[
            pltpu.VMEM((2, SCH), jnp.int32),
            pltpu.VMEM((2, SCH), jnp.int32),
            pltpu.VMEM((2, SCH), jnp.float32),
            pltpu.VMEM((2 * QCAP,), jnp.int32),
            pltpu.VMEM((2 * QCAP,), jnp.int32),
            pltpu.VMEM((2 * QCAP,), jnp.float32),
            pltpu.VMEM((2, GCH, b), jnp.float32),
            pltpu.VMEM((rpt, b), jnp.float32),
            pltpu.SemaphoreType.DMA,
            pltpu.SemaphoreType.DMA,
            pltpu.SemaphoreType.DMA,
            pltpu.SemaphoreType.DMA,
        ],
    )
    def sc_kernel(xT, rows_h, cols_h, vals_h, out,
                  rows_b, cols_b, vals_b, q_r, q_c, q_v,
                  gath, acc, sem_i0, sem_i1, sem_a, sem_b):
        cid = lax.axis_index("c")
        sid = lax.axis_index("s")
        wid = sid * NC + cid
        lo = wid * rpt

        def issue3(k, pp, sem):
            pltpu.async_copy(rows_h.at[k], rows_b.at[pp], sem)
            pltpu.async_copy(cols_h.at[k], cols_b.at[pp], sem)
            pltpu.async_copy(vals_h.at[k], vals_b.at[pp], sem)

        def wait3(k, pp, sem):
            pltpu.make_async_copy(rows_h.at[k], rows_b.at[pp], sem).wait()
            pltpu.make_async_copy(cols_h.at[k], cols_b.at[pp], sem).wait()
            pltpu.make_async_copy(vals_h.at[k], vals_b.at[pp], sem).wait()

        def issue_g(qp, d, p):
            src = xT.at[q_c.at[pl.ds(qp * QCAP + d * GCH, GCH)]]

            @pl.when(p == 0)
            def _():
                pltpu.async_copy(src, gath.at[0], sem_a)

            @pl.when(p == 1)
            def _():
                pltpu.async_copy(src, gath.at[1], sem_b)

        def wait_g(qp, d, p):
            src = xT.at[q_c.at[pl.ds(qp * QCAP + d * GCH, GCH)]]

            @pl.when(p == 0)
            def _():
                pltpu.make_async_copy(src, gath.at[0], sem_a).wait()

            @pl.when(p == 1)
            def _():
                pltpu.make_async_copy(src, gath.at[1], sem_b).wait()

        def consume(qp, d, p):
            def entrygrp(g, c2):
                qb = qp * QCAP + d * GCH + g * L
                rr = q_r[pl.ds(qb, L)]
                vv = q_v[pl.ds(qb, L)]
                for lane in range(L):
                    rl = rr[lane]
                    v = vv[lane]
                    i = g * L + lane
                    for m in range(b // L):
                        sl = pl.ds(m * L, L)
                        plsc.addupdate(acc.at[rl, sl], gath[p, i, sl] * v)
                return c2
            lax.fori_loop(0, GCH // L, entrygrp, 0)

        def drain_ready(qp, nready):
            def dbody(d, c2):
                p = d & 1

                @pl.when(d >= 2)
                def _():
                    issue_g(qp, d, p)
                wait_g(qp, d, p)
                consume(qp, d, p)
                return c2
            lax.fori_loop(0, nready, dbody, 0)

        def zrow(i, carry):
            for m in range(b // L):
                acc[i, pl.ds(m * L, L)] = jnp.zeros((L,), jnp.float32)
            return carry
        lax.fori_loop(0, rpt, zrow, 0)

        issue3(0, 0, sem_i0)

        def scan_chunk(k, carry):
            rem, nprev = carry
            pp = k & 1
            qo = 1 - pp

            pb = pp * QCAP
            ob = qo * QCAP + nprev * GCH
            for m in range(GCH // L):
                dst = pl.ds(pb + m * L, L)
                src = pl.ds(ob + m * L, L)
                q_r[dst] = q_r[src]
                q_c[dst] = q_c[src]
                q_v[dst] = q_v[src]

            @pl.when((k + 1 < nsc) & (pp == 0))
            def _():
                issue3(k + 1, 1, sem_i1)

            @pl.when((k + 1 < nsc) & (pp == 1))
            def _():
                issue3(k + 1, 0, sem_i0)

            @pl.when(pp == 0)
            def _():
                wait3(k, 0, sem_i0)

            @pl.when(pp == 1)
            def _():
                wait3(k, 1, sem_i1)

            def group(g8, cnt2):
                for u in range(8):
                    sl = pl.ds(g8 * (8 * L) + u * L, L)
                    r = rows_b[pp, sl]
                    msk = (r >= lo) & (r < lo + rpt)
                    npop = plsc.all_reduce_population_count(msk)[0]
                    cs = plsc.cumsum(jnp.where(msk, 1, 0))
                    idx = pb + cnt2 + cs - 1
                    plsc.store_scatter(q_r, [idx], r - lo, mask=msk)
                    plsc.store_scatter(q_c, [idx], cols_b[pp, sl], mask=msk)
                    plsc.store_scatter(q_v, [idx], vals_b[pp, sl], mask=msk)
                    cnt2 = cnt2 + npop
                return cnt2
            cnt = lax.fori_loop(0, SCH // (8 * L), group, rem)

            drain_ready(qo, nprev)

            nfull = cnt // GCH

            @pl.when(nfull > 0)
            def _():
                issue_g(pp, 0, 0)

            @pl.when(nfull > 1)
            def _():
                issue_g(pp, 1, 1)
            return (cnt - nfull * GCH, nfull)
        rem, nprev = lax.fori_loop(0, nsc, scan_chunk, (0, 0))

        lpp = (nsc - 1) & 1
        drain_ready(lpp, nprev)
        offt = lpp * QCAP + nprev * GCH
        for m in range(GCH // L):
            sl = pl.ds(offt + rem + m * L, L)
            q_r[sl] = jnp.zeros((L,), jnp.int32)
            q_c[sl] = jnp.zeros((L,), jnp.int32)
            q_v[sl] = jnp.zeros((L,), jnp.float32)
        pt = nprev & 1
        issue_g(lpp, nprev, pt)
        wait_g(lpp, nprev, pt)
        consume(lpp, nprev, pt)

        pltpu.sync_copy(acc, out.at[pl.ds(lo, rpt)])

    return sc_kernel


@functools.lru_cache(maxsize=None)
def _make_tc_epilogue(outf, b):
    blk = 512

    def body(p_ref, bias_ref, o_ref):
        o_ref[...] = p_ref[...].T + bias_ref[...]

    return pl.pallas_call(
        body,
        grid=(outf // blk,),
        in_specs=[
            pl.BlockSpec((blk, b), lambda i: (i, 0)),
            pl.BlockSpec((1, blk), lambda i: (0, i)),
        ],
        out_specs=pl.BlockSpec((b, blk), lambda i: (0, i)),
        out_shape=jax.ShapeDtypeStruct((b, outf), jnp.float32),
    )


def kernel(x, indices, values, bias):
    b, inf = x.shape
    outf = bias.shape[0]
    nnz = values.shape[0]

    rows = indices[0].astype(jnp.int32)
    cols = indices[1].astype(jnp.int32)
    vals = values.astype(jnp.float32)

    nsc = -(-nnz // SCH)
    pad = nsc * SCH - nnz
    if pad:
        rows = jnp.pad(rows, (0, pad))
        cols = jnp.pad(cols, (0, pad))
        vals = jnp.pad(vals, (0, pad))
    rows2 = rows.reshape(nsc, SCH)
    cols2 = cols.reshape(nsc, SCH)
    vals2 = vals.reshape(nsc, SCH)

    xT = x.T

    out_t = _make_sc_kernel(nsc, outf, b)(xT, rows2, cols2, vals2)
    return _make_tc_epilogue(outf, b)(out_t, bias.reshape(1, outf))


# speedup vs baseline: 1.6633x; 1.6633x over previous
import functools

import jax
import jax.numpy as jnp
from jax import lax
from jax.experimental import pallas as pl
from jax.experimental.pallas import tpu as pltpu
from jax.experimental.pallas import tpu_sc as plsc

NC = 2
NS = 16
L = 16
NW = NC * NS
SCH = 2048
GCH = 64
QCAP = SCH + 2 * GCH


@functools.lru_cache(maxsize=None)
def _make_sc_kernel(nsc, outf, b):
    rpt = outf // NW
    mesh = plsc.VectorSubcoreMesh(core_axis_name="c", subcore_axis_name="s")

    @functools.partial(
        pl.kernel,
        out_type=jax.ShapeDtypeStruct((outf, b), jnp.float32),
        mesh=mesh,
        compiler_params=pltpu.CompilerParams(needs_layout_passes=False),
        scratch_types=[
            pltpu.VMEM((2, SCH), jnp.int32),
            pltpu.VMEM((2, SCH), jnp.int32),
            pltpu.VMEM((2, SCH), jnp.float32),
            pltpu.VMEM((2 * QCAP,), jnp.int32),
            pltpu.VMEM((2 * QCAP,), jnp.int32),
            pltpu.VMEM((2 * QCAP,), jnp.float32),
            pltpu.VMEM((2, GCH, b), jnp.float32),
            pltpu.VMEM((rpt, b), jnp.float32),
            pltpu.SemaphoreType.DMA,
            pltpu.SemaphoreType.DMA,
            pltpu.SemaphoreType.DMA,
            pltpu.SemaphoreType.DMA,
        ],
    )
    def sc_kernel(xT, rows_h, cols_h, vals_h, out,
                  rows_b, cols_b, vals_b, q_r, q_c, q_v,
                  gath, acc, sem_i0, sem_i1, sem_a, sem_b):
        cid = lax.axis_index("c")
        sid = lax.axis_index("s")
        wid = sid * NC + cid
        lo = wid * rpt

        def issue3(k, pp, sem):
            pltpu.async_copy(rows_h.at[k], rows_b.at[pp], sem)
            pltpu.async_copy(cols_h.at[k], cols_b.at[pp], sem)
            pltpu.async_copy(vals_h.at[k], vals_b.at[pp], sem)

        def wait3(k, pp, sem):
            pltpu.make_async_copy(rows_h.at[k], rows_b.at[pp], sem).wait()
            pltpu.make_async_copy(cols_h.at[k], cols_b.at[pp], sem).wait()
            pltpu.make_async_copy(vals_h.at[k], vals_b.at[pp], sem).wait()

        def issue_g(qp, d, p):
            src = xT.at[q_c.at[pl.ds(qp * QCAP + d * GCH, GCH)]]

            @pl.when(p == 0)
            def _():
                pltpu.async_copy(src, gath.at[0], sem_a)

            @pl.when(p == 1)
            def _():
                pltpu.async_copy(src, gath.at[1], sem_b)

        def wait_g(qp, d, p):
            src = xT.at[q_c.at[pl.ds(qp * QCAP + d * GCH, GCH)]]

            @pl.when(p == 0)
            def _():
                pltpu.make_async_copy(src, gath.at[0], sem_a).wait()

            @pl.when(p == 1)
            def _():
                pltpu.make_async_copy(src, gath.at[1], sem_b).wait()

        def consume(qp, d, p):
            qbase = qp * QCAP + d * GCH

            @plsc.parallel_loop(0, GCH, unroll=2)
            def ent(e):
                qi = jnp.full((L,), qbase + e, jnp.int32)
                vs = plsc.load_gather(q_v, [qi])
                rl = plsc.load_gather(q_r, [qi])[0]
                for m in range(b // L):
                    sl = pl.ds(m * L, L)
                    plsc.addupdate(acc.at[rl, sl], gath[p, e, sl] * vs)

        def drain_ready(qp, nready):
            def dbody(d, c2):
                p = d & 1

                @pl.when(d >= 2)
                def _():
                    issue_g(qp, d, p)
                wait_g(qp, d, p)
                consume(qp, d, p)
                return c2
            lax.fori_loop(0, nready, dbody, 0)

        def zrow(i, carry):
            for m in range(b // L):
                acc[i, pl.ds(m * L, L)] = jnp.zeros((L,), jnp.float32)
            return carry
        lax.fori_loop(0, rpt, zrow, 0)

        issue3(0, 0, sem_i0)

        def scan_chunk(k, carry):
            rem, nprev = carry
            pp = k & 1
            qo = 1 - pp

            pb = pp * QCAP
            ob = qo * QCAP + nprev * GCH
            for m in range(GCH // L):
                dst = pl.ds(pb + m * L, L)
                src = pl.ds(ob + m * L, L)
                q_r[dst] = q_r[src]
                q_c[dst] = q_c[src]
                q_v[dst] = q_v[src]

            @pl.when((k + 1 < nsc) & (pp == 0))
            def _():
                issue3(k + 1, 1, sem_i1)

            @pl.when((k + 1 < nsc) & (pp == 1))
            def _():
                issue3(k + 1, 0, sem_i0)

            @pl.when(pp == 0)
            def _():
                wait3(k, 0, sem_i0)

            @pl.when(pp == 1)
            def _():
                wait3(k, 1, sem_i1)

            def group(g8, cnt2):
                for u in range(8):
                    sl = pl.ds(g8 * (8 * L) + u * L, L)
                    r = rows_b[pp, sl]
                    msk = (r >= lo) & (r < lo + rpt)
                    npop = plsc.all_reduce_population_count(msk)[0]
                    cs = plsc.cumsum(jnp.where(msk, 1, 0))
                    idx = pb + cnt2 + cs - 1
                    plsc.store_scatter(q_r, [idx], r - lo, mask=msk)
                    plsc.store_scatter(q_c, [idx], cols_b[pp, sl], mask=msk)
                    plsc.store_scatter(q_v, [idx], vals_b[pp, sl], mask=msk)
                    cnt2 = cnt2 + npop
                return cnt2
            cnt = lax.fori_loop(0, SCH // (8 * L), group, rem)

            drain_ready(qo, nprev)

            nfull = cnt // GCH

            @pl.when(nfull > 0)
            def _():
                issue_g(pp, 0, 0)

            @pl.when(nfull > 1)
            def _():
                issue_g(pp, 1, 1)
            return (cnt - nfull * GCH, nfull)
        rem, nprev = lax.fori_loop(0, nsc, scan_chunk, (0, 0))

        lpp = (nsc - 1) & 1
        drain_ready(lpp, nprev)
        offt = lpp * QCAP + nprev * GCH
        for m in range(GCH // L):
            sl = pl.ds(offt + rem + m * L, L)
            q_r[sl] = jnp.zeros((L,), jnp.int32)
            q_c[sl] = jnp.zeros((L,), jnp.int32)
            q_v[sl] = jnp.zeros((L,), jnp.float32)
        pt = nprev & 1
        issue_g(lpp, nprev, pt)
        wait_g(lpp, nprev, pt)
        consume(lpp, nprev, pt)

        pltpu.sync_copy(acc, out.at[pl.ds(lo, rpt)])

    return sc_kernel


@functools.lru_cache(maxsize=None)
def _make_tc_epilogue(outf, b):
    blk = 512

    def body(p_ref, bias_ref, o_ref):
        o_ref[...] = p_ref[...].T + bias_ref[...]

    return pl.pallas_call(
        body,
        grid=(outf // blk,),
        in_specs=[
            pl.BlockSpec((blk, b), lambda i: (i, 0)),
            pl.BlockSpec((1, blk), lambda i: (0, i)),
        ],
        out_specs=pl.BlockSpec((b, blk), lambda i: (0, i)),
        out_shape=jax.ShapeDtypeStruct((b, outf), jnp.float32),
    )


def kernel(x, indices, values, bias):
    b, inf = x.shape
    outf = bias.shape[0]
    nnz = values.shape[0]

    rows = indices[0].astype(jnp.int32)
    cols = indices[1].astype(jnp.int32)
    vals = values.astype(jnp.float32)

    nsc = -(-nnz // SCH)
    pad = nsc * SCH - nnz
    if pad:
        rows = jnp.pad(rows, (0, pad))
        cols = jnp.pad(cols, (0, pad))
        vals = jnp.pad(vals, (0, pad))
    rows2 = rows.reshape(nsc, SCH)
    cols2 = cols.reshape(nsc, SCH)
    vals2 = vals.reshape(nsc, SCH)

    xT = x.T

    out_t = _make_sc_kernel(nsc, outf, b)(xT, rows2, cols2, vals2)
    return _make_tc_epilogue(outf, b)(out_t, bias.reshape(1, outf))


# speedup vs baseline: 2.1706x; 1.3050x over previous
import functools

import jax
import jax.numpy as jnp
from jax import lax
from jax.experimental import pallas as pl
from jax.experimental.pallas import tpu as pltpu
from jax.experimental.pallas import tpu_sc as plsc

NC = 2
NS = 16
L = 16
NW = NC * NS
SCH = 2048
GCH = 64
QCAP = SCH + 2 * GCH


@functools.lru_cache(maxsize=None)
def _make_sc_kernel(nsc, outf, b):
    rpt = outf // NW
    mesh = plsc.VectorSubcoreMesh(core_axis_name="c", subcore_axis_name="s")

    @functools.partial(
        pl.kernel,
        out_type=jax.ShapeDtypeStruct((outf, b), jnp.float32),
        mesh=mesh,
        compiler_params=pltpu.CompilerParams(needs_layout_passes=False),
        scratch_types=[
            pltpu.VMEM((2, SCH), jnp.int32),
            pltpu.VMEM((2, SCH), jnp.int32),
            pltpu.VMEM((2, SCH), jnp.float32),
            pltpu.VMEM((2 * QCAP,), jnp.int32),
            pltpu.VMEM((2 * QCAP,), jnp.int32),
            pltpu.VMEM((2 * QCAP,), jnp.float32),
            pltpu.VMEM((2, GCH, b), jnp.float32),
            pltpu.VMEM((rpt, b), jnp.float32),
            pltpu.SemaphoreType.DMA,
            pltpu.SemaphoreType.DMA,
            pltpu.SemaphoreType.DMA,
            pltpu.SemaphoreType.DMA,
        ],
    )
    def sc_kernel(xT, rows_h, cols_h, vals_h, out,
                  rows_b, cols_b, vals_b, q_r, q_c, q_v,
                  gath, acc, sem_i0, sem_i1, sem_a, sem_b):
        cid = lax.axis_index("c")
        sid = lax.axis_index("s")
        wid = sid * NC + cid
        lo = wid * rpt

        def issue3(k, pp, sem):
            pltpu.async_copy(rows_h.at[k], rows_b.at[pp], sem)
            pltpu.async_copy(cols_h.at[k], cols_b.at[pp], sem)
            pltpu.async_copy(vals_h.at[k], vals_b.at[pp], sem)

        def wait3(k, pp, sem):
            pltpu.make_async_copy(rows_h.at[k], rows_b.at[pp], sem).wait()
            pltpu.make_async_copy(cols_h.at[k], cols_b.at[pp], sem).wait()
            pltpu.make_async_copy(vals_h.at[k], vals_b.at[pp], sem).wait()

        def issue_g(qp, d, p):
            src = xT.at[q_c.at[pl.ds(qp * QCAP + d * GCH, GCH)]]

            @pl.when(p == 0)
            def _():
                pltpu.async_copy(src, gath.at[0], sem_a)

            @pl.when(p == 1)
            def _():
                pltpu.async_copy(src, gath.at[1], sem_b)

        def wait_g(qp, d, p):
            src = xT.at[q_c.at[pl.ds(qp * QCAP + d * GCH, GCH)]]

            @pl.when(p == 0)
            def _():
                pltpu.make_async_copy(src, gath.at[0], sem_a).wait()

            @pl.when(p == 1)
            def _():
                pltpu.make_async_copy(src, gath.at[1], sem_b).wait()

        def consume(qp, d, p):
            qbase = qp * QCAP + d * GCH

            @plsc.parallel_loop(0, GCH, unroll=2)
            def ent(e):
                qi = jnp.full((L,), qbase + e, jnp.int32)
                vs = plsc.load_gather(q_v, [qi])
                rl = plsc.load_gather(q_r, [qi])[0]
                for m in range(b // L):
                    sl = pl.ds(m * L, L)
                    plsc.addupdate(acc.at[rl, sl], gath[p, e, sl] * vs)

        def drain_ready(qp, nready):
            def dbody(d, c2):
                p = d & 1

                @pl.when(d >= 2)
                def _():
                    issue_g(qp, d, p)
                wait_g(qp, d, p)
                consume(qp, d, p)
                return c2
            lax.fori_loop(0, nready, dbody, 0)

        def zrow(i, carry):
            for m in range(b // L):
                acc[i, pl.ds(m * L, L)] = jnp.zeros((L,), jnp.float32)
            return carry
        lax.fori_loop(0, rpt, zrow, 0)

        issue3(0, 0, sem_i0)

        def scan_chunk(k, carry):
            rem, nprev = carry
            pp = k & 1
            qo = 1 - pp

            pb = pp * QCAP
            ob = qo * QCAP + nprev * GCH
            for m in range(GCH // L):
                dst = pl.ds(pb + m * L, L)
                src = pl.ds(ob + m * L, L)
                q_r[dst] = q_r[src]
                q_c[dst] = q_c[src]
                q_v[dst] = q_v[src]

            @pl.when((k + 1 < nsc) & (pp == 0))
            def _():
                issue3(k + 1, 1, sem_i1)

            @pl.when((k + 1 < nsc) & (pp == 1))
            def _():
                issue3(k + 1, 0, sem_i0)

            @pl.when(pp == 0)
            def _():
                wait3(k, 0, sem_i0)

            @pl.when(pp == 1)
            def _():
                wait3(k, 1, sem_i1)

            @plsc.parallel_loop(0, SCH // L, unroll=4, carry=rem)
            def group(g, cnt2):
                sl = pl.ds(g * L, L)
                r = rows_b[pp, sl]
                msk = (r >= lo) & (r < lo + rpt)
                npop = plsc.all_reduce_population_count(msk)[0]
                cs = plsc.cumsum(jnp.where(msk, 1, 0))
                idx = pb + cnt2 + cs - 1
                plsc.store_scatter(q_r, [idx], r - lo, mask=msk)
                plsc.store_scatter(q_c, [idx], cols_b[pp, sl], mask=msk)
                plsc.store_scatter(q_v, [idx], vals_b[pp, sl], mask=msk)
                return cnt2 + npop
            cnt = group

            drain_ready(qo, nprev)

            nfull = cnt // GCH

            @pl.when(nfull > 0)
            def _():
                issue_g(pp, 0, 0)

            @pl.when(nfull > 1)
            def _():
                issue_g(pp, 1, 1)
            return (cnt - nfull * GCH, nfull)
        rem, nprev = lax.fori_loop(0, nsc, scan_chunk, (0, 0))

        lpp = (nsc - 1) & 1
        drain_ready(lpp, nprev)
        offt = lpp * QCAP + nprev * GCH
        for m in range(GCH // L):
            sl = pl.ds(offt + rem + m * L, L)
            q_r[sl] = jnp.zeros((L,), jnp.int32)
            q_c[sl] = jnp.zeros((L,), jnp.int32)
            q_v[sl] = jnp.zeros((L,), jnp.float32)
        pt = nprev & 1
        issue_g(lpp, nprev, pt)
        wait_g(lpp, nprev, pt)
        consume(lpp, nprev, pt)

        pltpu.sync_copy(acc, out.at[pl.ds(lo, rpt)])

    return sc_kernel


@functools.lru_cache(maxsize=None)
def _make_tc_epilogue(outf, b):
    blk = 512

    def body(p_ref, bias_ref, o_ref):
        o_ref[...] = p_ref[...].T + bias_ref[...]

    return pl.pallas_call(
        body,
        grid=(outf // blk,),
        in_specs=[
            pl.BlockSpec((blk, b), lambda i: (i, 0)),
            pl.BlockSpec((1, blk), lambda i: (0, i)),
        ],
        out_specs=pl.BlockSpec((b, blk), lambda i: (0, i)),
        out_shape=jax.ShapeDtypeStruct((b, outf), jnp.float32),
    )


def kernel(x, indices, values, bias):
    b, inf = x.shape
    outf = bias.shape[0]
    nnz = values.shape[0]

    rows = indices[0].astype(jnp.int32)
    cols = indices[1].astype(jnp.int32)
    vals = values.astype(jnp.float32)

    nsc = -(-nnz // SCH)
    pad = nsc * SCH - nnz
    if pad:
        rows = jnp.pad(rows, (0, pad))
        cols = jnp.pad(cols, (0, pad))
        vals = jnp.pad(vals, (0, pad))
    rows2 = rows.reshape(nsc, SCH)
    cols2 = cols.reshape(nsc, SCH)
    vals2 = vals.reshape(nsc, SCH)

    xT = x.T

    out_t = _make_sc_kernel(nsc, outf, b)(xT, rows2, cols2, vals2)
    return _make_tc_epilogue(outf, b)(out_t, bias.reshape(1, outf))
